# baseline (speedup 1.0000x reference)
import functools

import jax
import jax.numpy as jnp
from jax import lax
from jax.experimental import pallas as pl
from jax.experimental.pallas import tpu as pltpu
from jax.experimental.pallas import tpu_sc as plsc

N = 10000
E = 320000
D = 128
H = D // 2
EPS = 1e-5

NC = 2
NS = 16
EPT = E // NS
CH = 125
NCHUNK = EPT // CH
NPAD = 10112
RPT = NPAD // NS
NBUF = 8
AHEAD = 7


def _agg_body(with_cnt, *refs):
    if with_cnt:
        (xA, xB, src_hbm, dst_hbm, z2_hbm, z1_hbm,
         out_hbm, cnt_hbm,
         src_v, dst_v, b0, b1, b2, b3, b4, b5, b6, b7,
         s0, s1, s2, s3, s4, s5, s6, s7,
         ones_v, acc_sh, cnt_sh) = refs
    else:
        (xA, xB, src_hbm, dst_hbm, z2_hbm,
         out_hbm,
         src_v, dst_v, b0, b1, b2, b3, b4, b5, b6, b7,
         s0, s1, s2, s3, s4, s5, s6, s7,
         acc_sh) = refs
    bufs = [b0, b1, b2, b3, b4, b5, b6, b7]
    sems = [s0, s1, s2, s3, s4, s5, s6, s7]

    c = lax.axis_index("c")
    s = lax.axis_index("s")

    pltpu.sync_copy(src_hbm.at[s], src_v)
    if with_cnt:
        @pl.when(s == 0)
        def _():
            pltpu.sync_copy(z1_hbm, cnt_sh)

        def fill_ones(i, carry):
            ones_v[pl.ds(i * 16, 16)] = jnp.ones((16,), jnp.float32)
            return carry
        lax.fori_loop(0, 8, fill_ones, 0)

    def fire(j, b):
        @pl.when(c == 0)
        def _():
            pltpu.async_copy(xA.at[src_v.at[j]], bufs[b], sems[b])

        @pl.when(c == 1)
        def _():
            pltpu.async_copy(xB.at[src_v.at[j]], bufs[b], sems[b])

    for j0 in range(AHEAD):
        fire(j0, j0)

    pltpu.sync_copy(dst_hbm.at[s], dst_v)
    pltpu.sync_copy(z2_hbm.at[pl.ds(s * RPT, RPT)],
                    acc_sh.at[pl.ds(s * RPT, RPT)])

    plsc.subcore_barrier()

    def group(g, carry):
        for b in range(NBUF):
            j = g * NBUF + b
            nb = (b + AHEAD) % NBUF

            @pl.when(j + AHEAD < NCHUNK)
            def _():
                fire(j + AHEAD, nb)
            pltpu.make_async_copy(xA.at[src_v.at[j]], bufs[b],
                                  sems[b]).wait()
            pltpu.sync_copy(bufs[b], acc_sh.at[dst_v.at[j]], add=True)
            if with_cnt:
                @pl.when(c == jnp.int32(j >= NCHUNK // 2))
                def _():
                    pltpu.sync_copy(ones_v.at[pl.ds(0, CH)],
                                    cnt_sh.at[dst_v.at[j]], add=True)
        return carry
    lax.fori_loop(0, NCHUNK // NBUF, group, 0)

    plsc.subcore_barrier()

    pltpu.sync_copy(acc_sh.at[pl.ds(s * RPT, RPT)],
                    out_hbm.at[c, pl.ds(s * RPT, RPT)])
    if with_cnt:
        @pl.when(s == 0)
        def _():
            pltpu.sync_copy(cnt_sh, cnt_hbm.at[c])


def _make_agg(with_cnt):
    out_type = [jax.ShapeDtypeStruct((NC, NPAD, H), jnp.bfloat16)]
    scratch = [
        pltpu.VMEM((NCHUNK, CH), jnp.int32),
        pltpu.VMEM((NCHUNK, CH), jnp.int32),
    ]
    scratch += [pltpu.VMEM((CH, H), jnp.bfloat16) for _ in range(NBUF)]
    scratch += [pltpu.SemaphoreType.DMA for _ in range(NBUF)]
    if with_cnt:
        out_type.append(jax.ShapeDtypeStruct((NC, N), jnp.float32))
        scratch.append(pltpu.VMEM((128,), jnp.float32))
    scratch.append(pltpu.VMEM_SHARED((NPAD, H), jnp.bfloat16))
    if with_cnt:
        scratch.append(pltpu.VMEM_SHARED((N,), jnp.float32))
    mesh = plsc.VectorSubcoreMesh(core_axis_name="c", subcore_axis_name="s")
    return pl.kernel(
        functools.partial(_agg_body, with_cnt),
        out_type=out_type,
        mesh=mesh,
        scratch_types=scratch,
        compiler_params=pltpu.CompilerParams(use_tc_tiling_on_sc=False),
    )


_CONTRACT = (((1,), (1,)), ((), ()))


_DOT = functools.partial(lax.dot_general, dimension_numbers=_CONTRACT,
                         preferred_element_type=jnp.float32)


def _mean(P_ref, cnt_ref):
    cnt = jnp.sum(cnt_ref[...], axis=1, keepdims=True)
    inv = 1.0 / jnp.maximum(cnt, 1.0)
    return P_ref[0].astype(jnp.float32) * inv, P_ref[1].astype(jnp.float32) * inv


def _norm_relu(h, g_ref, be_ref):
    mu = jnp.mean(h, axis=1, keepdims=True)
    var = jnp.mean((h - mu) ** 2, axis=1, keepdims=True)
    h = (h - mu) / jnp.sqrt(var + EPS) * g_ref[...] + be_ref[...]
    return jnp.maximum(h, 0.0)


def _tc1_body(P_ref, cnt_ref, x_ref, WlA_ref, WlB_ref, bl_ref,
              Wr_ref, g_ref, be_ref, oA_ref, oB_ref):
    mA, mB = _mean(P_ref, cnt_ref)
    h = (_DOT(mA, WlA_ref[...]) + _DOT(mB, WlB_ref[...]) + bl_ref[...]
         + _DOT(x_ref[...], Wr_ref[...]))
    h = _norm_relu(h, g_ref, be_ref)
    oA_ref[...] = h[:, :H].astype(jnp.bfloat16)
    oB_ref[...] = h[:, H:].astype(jnp.bfloat16)


def _tc2_body(P_ref, cnt_ref, hA_ref, hB_ref, WlA_ref, WlB_ref, bl_ref,
              WrA_ref, WrB_ref, g_ref, be_ref, Wfc_ref, bfc_ref, o_ref):
    mA, mB = _mean(P_ref, cnt_ref)
    h = (_DOT(mA, WlA_ref[...]) + _DOT(mB, WlB_ref[...]) + bl_ref[...]
         + _DOT(hA_ref[...].astype(jnp.float32), WrA_ref[...])
         + _DOT(hB_ref[...].astype(jnp.float32), WrB_ref[...]))
    h = _norm_relu(h, g_ref, be_ref)
    o_ref[...] = jnp.sum(h * Wfc_ref[...], axis=1, keepdims=True) + bfc_ref[0, 0]


_R = 1000

_PCNT = [
    pl.BlockSpec((NC, _R, H), lambda i: (0, i, 0)),
    pl.BlockSpec((_R, NC), lambda i: (i, 0)),
]
_WFULL = pl.BlockSpec((D, D), lambda i: (0, 0))
_WHALF = pl.BlockSpec((D, H), lambda i: (0, 0))
_VROW = pl.BlockSpec((1, D), lambda i: (0, 0))
_XHALF = pl.BlockSpec((_R, H), lambda i: (i, 0))


def _tc_layer1(*args):
    return pl.pallas_call(
        _tc1_body,
        grid=(N // _R,),
        in_specs=_PCNT + [
            pl.BlockSpec((_R, D), lambda i: (i, 0)),
            _WHALF, _WHALF, _VROW, _WFULL, _VROW, _VROW,
        ],
        out_specs=[_XHALF, _XHALF],
        out_shape=[jax.ShapeDtypeStruct((N, H), jnp.bfloat16),
                   jax.ShapeDtypeStruct((N, H), jnp.bfloat16)],
    )(*args)


def _tc_layer2(*args):
    return pl.pallas_call(
        _tc2_body,
        grid=(N // _R,),
        in_specs=_PCNT + [
            _XHALF, _XHALF,
            _WHALF, _WHALF, _VROW, _WHALF, _WHALF, _VROW, _VROW,
            _VROW,
            pl.BlockSpec((1, 1), lambda i: (0, 0)),
        ],
        out_specs=pl.BlockSpec((_R, 1), lambda i: (i, 0)),
        out_shape=jax.ShapeDtypeStruct((N, 1), jnp.float32),
    )(*args)


def kernel(x, edge_index, W_l1, b_l1, W_r1, W_l2, b_l2, W_r2,
           g1, be1, g2, be2, W_fc, b_fc):
    src = edge_index[0].reshape(NS, NCHUNK, CH)
    dst = edge_index[1].reshape(NS, NCHUNK, CH)
    z2 = jnp.zeros((NPAD, H), jnp.bfloat16)
    z1 = jnp.zeros((N,), jnp.float32)
    xbA = x[:, :H].astype(jnp.bfloat16)
    xbB = x[:, H:].astype(jnp.bfloat16)

    agg1 = _make_agg(True)
    agg2 = _make_agg(False)

    P1, cnt = agg1(xbA, xbB, src, dst, z2, z1)
    cnt1 = cnt.T
    h1A, h1B = _tc_layer1(
        P1, cnt1, x, W_l1[:, :H], W_l1[:, H:], b_l1.reshape(1, D),
        W_r1, g1.reshape(1, D), be1.reshape(1, D))
    (P2,) = agg2(h1A, h1B, src, dst, z2)
    out = _tc_layer2(
        P2, cnt1, h1A, h1B, W_l2[:, :H], W_l2[:, H:], b_l2.reshape(1, D),
        W_r2[:, :H], W_r2[:, H:], g2.reshape(1, D), be2.reshape(1, D),
        W_fc, b_fc.reshape(1, 1))
    return out.reshape(N)


# speedup vs baseline: 1.0562x; 1.0562x over previous
import functools

import jax
import jax.numpy as jnp
from jax import lax
from jax.experimental import pallas as pl
from jax.experimental.pallas import tpu as pltpu
from jax.experimental.pallas import tpu_sc as plsc

N = 10000
E = 320000
D = 128
H = D // 2
EPS = 1e-5

NC = 2
NS = 16
EPT = E // NS
CH = 125
NCHUNK = EPT // CH
NPAD = 10112
RPT = NPAD // NS
NBUF = 8
AHEAD = 7


def _agg_body(with_cnt, *refs):
    if with_cnt:
        (xA, xB, src_hbm, dst_hbm, z2_hbm, z1_hbm,
         out_hbm, cnt_hbm,
         src_v, dst_v, b0, b1, b2, b3, b4, b5, b6, b7,
         s0, s1, s2, s3, s4, s5, s6, s7,
         ones_v, acc_sh, cnt_sh) = refs
    else:
        (xA, xB, src_hbm, dst_hbm, z2_hbm,
         out_hbm,
         src_v, dst_v, b0, b1, b2, b3, b4, b5, b6, b7,
         s0, s1, s2, s3, s4, s5, s6, s7,
         acc_sh) = refs
    bufs = [b0, b1, b2, b3, b4, b5, b6, b7]
    sems = [s0, s1, s2, s3, s4, s5, s6, s7]

    c = lax.axis_index("c")
    s = lax.axis_index("s")

    pltpu.sync_copy(src_hbm.at[s], src_v)
    if with_cnt:
        @pl.when(s == 0)
        def _():
            pltpu.sync_copy(z1_hbm, cnt_sh)

        def fill_ones(i, carry):
            ones_v[pl.ds(i * 16, 16)] = jnp.ones((16,), jnp.float32)
            return carry
        lax.fori_loop(0, 8, fill_ones, 0)

    def fire(j, b):
        @pl.when(c == 0)
        def _():
            pltpu.async_copy(xA.at[src_v.at[j]], bufs[b], sems[b])

        @pl.when(c == 1)
        def _():
            pltpu.async_copy(xB.at[src_v.at[j]], bufs[b], sems[b])

    for j0 in range(AHEAD):
        fire(j0, j0)

    pltpu.sync_copy(dst_hbm.at[s], dst_v)
    pltpu.sync_copy(z2_hbm.at[pl.ds(s * RPT, RPT)],
                    acc_sh.at[pl.ds(s * RPT, RPT)])

    plsc.subcore_barrier()

    def group(g, carry):
        for b in range(NBUF):
            j = g * NBUF + b
            nb = (b + AHEAD) % NBUF

            @pl.when(j + AHEAD < NCHUNK)
            def _():
                fire(j + AHEAD, nb)
            pltpu.make_async_copy(xA.at[src_v.at[j]], bufs[b],
                                  sems[b]).wait()
            pltpu.sync_copy(bufs[b], acc_sh.at[dst_v.at[j]], add=True)
            if with_cnt:
                @pl.when(c == jnp.int32(j >= NCHUNK // 2))
                def _():
                    pltpu.sync_copy(ones_v.at[pl.ds(0, CH)],
                                    cnt_sh.at[dst_v.at[j]], add=True)
        return carry
    lax.fori_loop(0, NCHUNK // NBUF, group, 0)

    plsc.subcore_barrier()

    pltpu.sync_copy(acc_sh.at[pl.ds(s * RPT, RPT)],
                    out_hbm.at[c, pl.ds(s * RPT, RPT)])
    if with_cnt:
        @pl.when(s == 0)
        def _():
            pltpu.sync_copy(cnt_sh, cnt_hbm.at[c])


def _make_agg(with_cnt):
    out_type = [jax.ShapeDtypeStruct((NC, NPAD, H), jnp.bfloat16)]
    scratch = [
        pltpu.VMEM((NCHUNK, CH), jnp.int32),
        pltpu.VMEM((NCHUNK, CH), jnp.int32),
    ]
    scratch += [pltpu.VMEM((CH, H), jnp.bfloat16) for _ in range(NBUF)]
    scratch += [pltpu.SemaphoreType.DMA for _ in range(NBUF)]
    if with_cnt:
        out_type.append(jax.ShapeDtypeStruct((NC, N), jnp.float32))
        scratch.append(pltpu.VMEM((128,), jnp.float32))
    scratch.append(pltpu.VMEM_SHARED((NPAD, H), jnp.bfloat16))
    if with_cnt:
        scratch.append(pltpu.VMEM_SHARED((N,), jnp.float32))
    mesh = plsc.VectorSubcoreMesh(core_axis_name="c", subcore_axis_name="s")
    return pl.kernel(
        functools.partial(_agg_body, with_cnt),
        out_type=out_type,
        mesh=mesh,
        scratch_types=scratch,
        compiler_params=pltpu.CompilerParams(use_tc_tiling_on_sc=False),
    )


_CONTRACT = (((1,), (1,)), ((), ()))


_DOT = functools.partial(lax.dot_general, dimension_numbers=_CONTRACT,
                         preferred_element_type=jnp.float32)


def _mean(P_ref, cnt_ref):
    cnt = jnp.sum(cnt_ref[...], axis=1, keepdims=True)
    inv = 1.0 / jnp.maximum(cnt, 1.0)
    return P_ref[0].astype(jnp.float32) * inv, P_ref[1].astype(jnp.float32) * inv


def _norm_relu(h, g_ref, be_ref):
    mu = jnp.mean(h, axis=1, keepdims=True)
    var = jnp.mean((h - mu) ** 2, axis=1, keepdims=True)
    h = (h - mu) / jnp.sqrt(var + EPS) * g_ref[...] + be_ref[...]
    return jnp.maximum(h, 0.0)


def _tc1_body(P_ref, cnt_ref, x_ref, WlA_ref, WlB_ref, bl_ref,
              Wr_ref, g_ref, be_ref, oA_ref, oB_ref):
    mA, mB = _mean(P_ref, cnt_ref)
    h = (_DOT(mA, WlA_ref[...]) + _DOT(mB, WlB_ref[...]) + bl_ref[...]
         + _DOT(x_ref[...], Wr_ref[...]))
    h = _norm_relu(h, g_ref, be_ref)
    oA_ref[...] = h[:, :H].astype(jnp.bfloat16)
    oB_ref[...] = h[:, H:].astype(jnp.bfloat16)


def _tc2_body(P_ref, cnt_ref, hA_ref, hB_ref, WlA_ref, WlB_ref, bl_ref,
              WrA_ref, WrB_ref, g_ref, be_ref, Wfc_ref, bfc_ref, o_ref):
    mA, mB = _mean(P_ref, cnt_ref)
    h = (_DOT(mA, WlA_ref[...]) + _DOT(mB, WlB_ref[...]) + bl_ref[...]
         + _DOT(hA_ref[...].astype(jnp.float32), WrA_ref[...])
         + _DOT(hB_ref[...].astype(jnp.float32), WrB_ref[...]))
    h = _norm_relu(h, g_ref, be_ref)
    o_ref[...] = jnp.sum(h * Wfc_ref[...], axis=1, keepdims=True) + bfc_ref[0, 0]


_R = 2000

_PCNT = [
    pl.BlockSpec((NC, _R, H), lambda i: (0, i, 0)),
    pl.BlockSpec((_R, NC), lambda i: (i, 0)),
]
_WFULL = pl.BlockSpec((D, D), lambda i: (0, 0))
_WHALF = pl.BlockSpec((D, H), lambda i: (0, 0))
_VROW = pl.BlockSpec((1, D), lambda i: (0, 0))
_XHALF = pl.BlockSpec((_R, H), lambda i: (i, 0))


def _tc_layer1(*args):
    return pl.pallas_call(
        _tc1_body,
        grid=(N // _R,),
        in_specs=_PCNT + [
            pl.BlockSpec((_R, D), lambda i: (i, 0)),
            _WHALF, _WHALF, _VROW, _WFULL, _VROW, _VROW,
        ],
        out_specs=[_XHALF, _XHALF],
        out_shape=[jax.ShapeDtypeStruct((N, H), jnp.bfloat16),
                   jax.ShapeDtypeStruct((N, H), jnp.bfloat16)],
    )(*args)


def _tc_layer2(*args):
    return pl.pallas_call(
        _tc2_body,
        grid=(N // _R,),
        in_specs=_PCNT + [
            _XHALF, _XHALF,
            _WHALF, _WHALF, _VROW, _WHALF, _WHALF, _VROW, _VROW,
            _VROW,
            pl.BlockSpec((1, 1), lambda i: (0, 0)),
        ],
        out_specs=pl.BlockSpec((_R, 1), lambda i: (i, 0)),
        out_shape=jax.ShapeDtypeStruct((N, 1), jnp.float32),
    )(*args)


def kernel(x, edge_index, W_l1, b_l1, W_r1, W_l2, b_l2, W_r2,
           g1, be1, g2, be2, W_fc, b_fc):
    src = edge_index[0].reshape(NS, NCHUNK, CH)
    dst = edge_index[1].reshape(NS, NCHUNK, CH)
    z2 = jnp.zeros((NPAD, H), jnp.bfloat16)
    z1 = jnp.zeros((N,), jnp.float32)
    xbA = x[:, :H].astype(jnp.bfloat16)
    xbB = x[:, H:].astype(jnp.bfloat16)

    agg1 = _make_agg(True)
    agg2 = _make_agg(False)

    P1, cnt = agg1(xbA, xbB, src, dst, z2, z1)
    cnt1 = cnt.T
    h1A, h1B = _tc_layer1(
        P1, cnt1, x, W_l1[:, :H], W_l1[:, H:], b_l1.reshape(1, D),
        W_r1, g1.reshape(1, D), be1.reshape(1, D))
    (P2,) = agg2(h1A, h1B, src, dst, z2)
    out = _tc_layer2(
        P2, cnt1, h1A, h1B, W_l2[:, :H], W_l2[:, H:], b_l2.reshape(1, D),
        W_r2[:, :H], W_r2[:, H:], g2.reshape(1, D), be2.reshape(1, D),
        W_fc, b_fc.reshape(1, 1))
    return out.reshape(N)
